# baseline (speedup 1.0000x reference)
import functools
import jax
import jax.numpy as jnp
from jax import lax
from jax.experimental import pallas as pl
from jax.experimental.pallas import tpu as pltpu
from jax.experimental.pallas import tpu_sc as plsc

_L = 16
_NS = 16
_NC = 2
_NP = 2
_NQ = _NC * _NP
_K = 80
_DT = 10
_DR = 1000
_ZCH = 200


def _node_proj_body(x_ref, w4_ref, b4_ref, esrc_ref, edst_ref, bh_ref, cx_ref):
    x = x_ref[...]
    w = w4_ref[...]
    b = b4_ref[...]
    esrc_ref[...] = jnp.dot(x, w[0], preferred_element_type=jnp.float32) + b[0]
    edst_ref[...] = jnp.dot(x, w[1], preferred_element_type=jnp.float32) + b[1]
    bh_ref[...] = jnp.dot(x, w[2], preferred_element_type=jnp.float32) + b[2]
    cx_ref[...] = jnp.dot(x, w[3], preferred_element_type=jnp.float32) + b[3]


def _node_proj(node_feats, Ws, bs, block_rows=2000):
    n, d = node_feats.shape
    w4 = jnp.stack(Ws)
    b4 = jnp.stack(bs)
    grid = (n // block_rows,)
    out = jax.ShapeDtypeStruct((n, d), jnp.float32)
    return pl.pallas_call(
        _node_proj_body,
        grid=grid,
        in_specs=[
            pl.BlockSpec((block_rows, d), lambda i: (i, 0)),
            pl.BlockSpec((4, d, d), lambda i: (0, 0, 0)),
            pl.BlockSpec((4, d), lambda i: (0, 0)),
        ],
        out_specs=[pl.BlockSpec((block_rows, d), lambda i: (i, 0))] * 4,
        out_shape=[out, out, out, out],
    )(node_feats, w4, b4)


def _edge_proj_body(x_ref, w_ref, b_ref, o_ref):
    res = (
        jnp.dot(x_ref[...], w_ref[...], preferred_element_type=jnp.float32)
        + b_ref[...]
    )
    w32 = res.shape[-1] // _NQ
    for q in range(_NQ):
        o_ref[q, :, :] = res[:, q * w32:(q + 1) * w32]


def _edge_proj(edge_feats, W, b, block_rows=4000):
    e, d = edge_feats.shape
    w32 = d // _NQ
    grid = (e // block_rows,)
    return pl.pallas_call(
        _edge_proj_body,
        grid=grid,
        in_specs=[
            pl.BlockSpec((block_rows, d), lambda i: (i, 0)),
            pl.BlockSpec((d, d), lambda i: (0, 0)),
            pl.BlockSpec((1, d), lambda i: (0, 0)),
        ],
        out_specs=pl.BlockSpec((_NQ, block_rows, w32), lambda i: (0, i, 0)),
        out_shape=jax.ShapeDtypeStruct((_NQ, e, w32), jnp.float32),
    )(edge_feats, W, b.reshape(1, d))


def _sc_middle(esrc4, edst4, bh4, g4, src, dst, n, e, w):
    rows_per_tile = n // _NS
    edges_per_tile = e // _NS
    chunks = edges_per_tile // _K
    cols = w // _L

    mesh = plsc.VectorSubcoreMesh(core_axis_name="c", subcore_axis_name="s")

    @functools.partial(
        pl.kernel,
        out_type=[
            jax.ShapeDtypeStruct((_NQ * e, w), jnp.float32),
            jax.ShapeDtypeStruct((_NQ * n, 2 * w), jnp.float32),
        ],
        mesh=mesh,
        compiler_params=pltpu.CompilerParams(use_tc_tiling_on_sc=False),
        scratch_types=[
            pltpu.VMEM((_K,), jnp.int32),
            pltpu.VMEM((_K,), jnp.int32),
            pltpu.VMEM((_K,), jnp.int32),
            pltpu.VMEM((_K,), jnp.int32),
            pltpu.VMEM((_K, w), jnp.float32),
            pltpu.VMEM((_K, w), jnp.float32),
            pltpu.VMEM((_K, w), jnp.float32),
            pltpu.VMEM((_K, w), jnp.float32),
            pltpu.VMEM((_K, 2 * w), jnp.float32),
            pltpu.VMEM((_ZCH, 2 * w), jnp.float32),
            pltpu.VMEM_SHARED((n, 2 * w), jnp.float32),
            pltpu.SemaphoreType.DMA,
            pltpu.SemaphoreType.DMA,
            pltpu.SemaphoreType.DMA,
        ],
    )
    def sc_kernel(esrc_hbm, edst_hbm, bh_hbm, g_hbm, src_hbm, dst_hbm,
                  m_hbm, sums_hbm,
                  src_i, dst_i, isrc, idst, a_s, a_d, a_g, a_bh, comb, zbuf,
                  acc, sem1, sem2, sem3):
        c = lax.axis_index("c")
        s = lax.axis_index("s")

        def zero_body(r, _):
            for col in range(2 * cols):
                zbuf[r, pl.ds(col * _L, _L)] = jnp.zeros((_L,), jnp.float32)
            return 0
        lax.fori_loop(0, _ZCH, zero_body, 0)

        base = s * edges_per_tile

        for p in range(_NP):
            q = p * _NC + c

            @pl.when(s < _DT)
            def _():
                for i in range(_DR // _ZCH):
                    r0 = s * _DR + i * _ZCH
                    pltpu.sync_copy(zbuf, acc.at[pl.ds(r0, _ZCH)])
            plsc.subcore_barrier()

            def chunk_body(j, _):
                e0 = base + j * _K
                pltpu.sync_copy(src_hbm.at[pl.ds(e0, _K)], src_i)
                pltpu.sync_copy(dst_hbm.at[pl.ds(e0, _K)], dst_i)
                for v in range(_K // _L):
                    sl = pl.ds(v * _L, _L)
                    isrc[sl] = src_i[sl] * _NQ + q
                    idst[sl] = dst_i[sl] * _NQ + q
                d1 = pltpu.async_copy(esrc_hbm.at[isrc], a_s, sem1)
                d2 = pltpu.async_copy(edst_hbm.at[idst], a_d, sem2)
                d3 = pltpu.async_copy(bh_hbm.at[isrc], a_bh, sem3)
                pltpu.sync_copy(g_hbm.at[pl.ds(q * e + e0, _K)], a_g)
                d1.wait()
                d2.wait()
                d3.wait()

                def row_body(r, _):
                    for col in range(cols):
                        sl = pl.ds(col * _L, _L)
                        mv = a_s[r, sl] + a_d[r, sl] + a_g[r, sl]
                        sig = 1.0 / (1.0 + jnp.exp(-mv))
                        a_g[r, sl] = mv
                        comb[r, pl.ds(w + col * _L, _L)] = sig
                        comb[r, sl] = sig * a_bh[r, sl]
                    return 0
                lax.fori_loop(0, _K, row_body, 0)

                pltpu.sync_copy(a_g, m_hbm.at[pl.ds(q * e + e0, _K)])
                pltpu.sync_copy(comb, acc.at[dst_i], add=True)
                return 0

            lax.fori_loop(0, chunks, chunk_body, 0)

            plsc.subcore_barrier()

            @pl.when(s < _DT)
            def _():
                r0 = s * _DR
                pltpu.sync_copy(acc.at[pl.ds(r0, _DR)],
                                sums_hbm.at[pl.ds(q * n + r0, _DR)])
            plsc.subcore_barrier()

    return sc_kernel(esrc4, edst4, bh4, g4, src, dst)


def _ln_silu(v, gamma, beta):
    mu = jnp.mean(v, axis=-1, keepdims=True)
    var = jnp.mean(jnp.square(v - mu), axis=-1, keepdims=True)
    t = (v - mu) * jax.lax.rsqrt(var + 1e-5) * gamma + beta
    return t * jax.nn.sigmoid(t)


def _edge_final_body(m_ref, ef_ref, g_ref, bt_ref, y_ref):
    m = jnp.concatenate([m_ref[q] for q in range(_NQ)], axis=-1)
    y_ref[...] = ef_ref[...] + _ln_silu(m, g_ref[...], bt_ref[...])


def _edge_final(m4, edge_feats, gamma, beta, block_rows=4000):
    e, d = edge_feats.shape
    w32 = d // _NQ
    grid = (e // block_rows,)
    return pl.pallas_call(
        _edge_final_body,
        grid=grid,
        in_specs=[
            pl.BlockSpec((_NQ, block_rows, w32), lambda i: (0, i, 0)),
            pl.BlockSpec((block_rows, d), lambda i: (i, 0)),
            pl.BlockSpec((1, d), lambda i: (0, 0)),
            pl.BlockSpec((1, d), lambda i: (0, 0)),
        ],
        out_specs=pl.BlockSpec((block_rows, d), lambda i: (i, 0)),
        out_shape=jax.ShapeDtypeStruct((e, d), jnp.float32),
    )(m4, edge_feats, gamma.reshape(1, d), beta.reshape(1, d))


def _node_final_body(cx_ref, sums_ref, nf_ref, g_ref, bt_ref, x_ref):
    w32 = sums_ref.shape[-1] // 2
    ssh = jnp.concatenate([sums_ref[q, :, :w32] for q in range(_NQ)], axis=-1)
    ss = jnp.concatenate([sums_ref[q, :, w32:] for q in range(_NQ)], axis=-1)
    h = ssh / (ss + 1e-6)
    v = cx_ref[...] + h
    x_ref[...] = nf_ref[...] + _ln_silu(v, g_ref[...], bt_ref[...])


def _node_final(cx, sums4, node_feats, gamma, beta, block_rows=2000):
    n, d = cx.shape
    grid = (n // block_rows,)
    blk = pl.BlockSpec((block_rows, d), lambda i: (i, 0))
    vec = pl.BlockSpec((1, d), lambda i: (0, 0))
    return pl.pallas_call(
        _node_final_body,
        grid=grid,
        in_specs=[
            blk,
            pl.BlockSpec((_NQ, block_rows, sums4.shape[-1]),
                         lambda i: (0, i, 0)),
            blk, vec, vec,
        ],
        out_specs=blk,
        out_shape=jax.ShapeDtypeStruct((n, d), jnp.float32),
    )(cx, sums4, node_feats, gamma.reshape(1, d), beta.reshape(1, d))


def kernel(node_feats, edge_feats, edge_index,
           W_src_gate, b_src_gate, W_dst_gate, b_dst_gate,
           W_edge_gate, b_edge_gate, W_src_update, b_src_update,
           W_dst_update, b_dst_update,
           gamma_nodes, beta_nodes, gamma_edges, beta_edges):
    n, d = node_feats.shape
    w = d // _NQ
    src = edge_index[0]
    dst = edge_index[1]

    e_src, e_dst, bh, cx = _node_proj(
        node_feats,
        [W_src_gate, W_dst_gate, W_dst_update, W_src_update],
        [b_src_gate, b_dst_gate, b_dst_update, b_src_update],
    )
    g4 = _edge_proj(edge_feats, W_edge_gate, b_edge_gate)
    e = g4.shape[1]

    esrc4 = e_src.reshape(n * _NQ, w)
    edst4 = e_dst.reshape(n * _NQ, w)
    bh4 = bh.reshape(n * _NQ, w)
    m_flat, sums_flat = _sc_middle(
        esrc4, edst4, bh4, g4.reshape(_NQ * e, w), src, dst, n, e, w)
    m4 = m_flat.reshape(_NQ, e, w)
    sums4 = sums_flat.reshape(_NQ, n, 2 * w)

    y = _edge_final(m4, edge_feats, gamma_edges, beta_edges)
    x = _node_final(cx, sums4, node_feats, gamma_nodes, beta_nodes)
    return (x, y)


# speedup vs baseline: 1.2474x; 1.2474x over previous
import functools
import jax
import jax.numpy as jnp
from jax import lax
from jax.experimental import pallas as pl
from jax.experimental.pallas import tpu as pltpu
from jax.experimental.pallas import tpu_sc as plsc

_L = 16
_NS = 16
_NC = 2
_NP = 2
_NQ = _NC * _NP
_K = 80
_DT = 10
_DR = 1000
_ZCH = 200


def _node_proj_body(x_ref, w4_ref, b4_ref, esrc_ref, edst_ref, bh_ref, cx_ref):
    x = x_ref[...]
    w = w4_ref[...]
    b = b4_ref[...]
    esrc_ref[...] = jnp.dot(x, w[0], preferred_element_type=jnp.float32) + b[0]
    edst_ref[...] = jnp.dot(x, w[1], preferred_element_type=jnp.float32) + b[1]
    bh_ref[...] = jnp.dot(x, w[2], preferred_element_type=jnp.float32) + b[2]
    cx_ref[...] = jnp.dot(x, w[3], preferred_element_type=jnp.float32) + b[3]


def _node_proj(node_feats, Ws, bs, block_rows=2000):
    n, d = node_feats.shape
    w4 = jnp.stack(Ws)
    b4 = jnp.stack(bs)
    grid = (n // block_rows,)
    out = jax.ShapeDtypeStruct((n, d), jnp.float32)
    return pl.pallas_call(
        _node_proj_body,
        grid=grid,
        in_specs=[
            pl.BlockSpec((block_rows, d), lambda i: (i, 0)),
            pl.BlockSpec((4, d, d), lambda i: (0, 0, 0)),
            pl.BlockSpec((4, d), lambda i: (0, 0)),
        ],
        out_specs=[pl.BlockSpec((block_rows, d), lambda i: (i, 0))] * 4,
        out_shape=[out, out, out, out],
    )(node_feats, w4, b4)


def _edge_proj_body(x_ref, w_ref, b_ref, o_ref):
    res = (
        jnp.dot(x_ref[...], w_ref[...], preferred_element_type=jnp.float32)
        + b_ref[...]
    )
    w32 = res.shape[-1] // _NQ
    for q in range(_NQ):
        o_ref[q, :, :] = res[:, q * w32:(q + 1) * w32]


def _edge_proj(edge_feats, W, b, block_rows=4000):
    e, d = edge_feats.shape
    w32 = d // _NQ
    grid = (e // block_rows,)
    return pl.pallas_call(
        _edge_proj_body,
        grid=grid,
        in_specs=[
            pl.BlockSpec((block_rows, d), lambda i: (i, 0)),
            pl.BlockSpec((d, d), lambda i: (0, 0)),
            pl.BlockSpec((1, d), lambda i: (0, 0)),
        ],
        out_specs=pl.BlockSpec((_NQ, block_rows, w32), lambda i: (0, i, 0)),
        out_shape=jax.ShapeDtypeStruct((_NQ, e, w32), jnp.float32),
    )(edge_feats, W, b.reshape(1, d))


def _sc_middle(esrc4, edst4, bh4, g4, pk, n, e, w):
    ept = e // _NS
    chunks = ept // _K
    cols = w // _L

    mesh = plsc.VectorSubcoreMesh(core_axis_name="c", subcore_axis_name="s")

    @functools.partial(
        pl.kernel,
        out_type=[
            jax.ShapeDtypeStruct((_NQ * e, w), jnp.float32),
            jax.ShapeDtypeStruct((_NQ * n, 2 * w), jnp.float32),
        ],
        mesh=mesh,
        compiler_params=pltpu.CompilerParams(use_tc_tiling_on_sc=False),
        scratch_types=[
            pltpu.VMEM((ept,), jnp.int32),
            [pltpu.VMEM((_K,), jnp.int32)] * 2,
            [pltpu.VMEM((_K,), jnp.int32)] * 2,
            [pltpu.VMEM((_K,), jnp.int32)] * 2,
            [pltpu.VMEM((_K, w), jnp.float32)] * 2,
            [pltpu.VMEM((_K, w), jnp.float32)] * 2,
            [pltpu.VMEM((_K, w), jnp.float32)] * 2,
            [pltpu.VMEM((_K, w), jnp.float32)] * 2,
            [pltpu.VMEM((_K, 2 * w), jnp.float32)] * 2,
            pltpu.VMEM((_ZCH, 2 * w), jnp.float32),
            pltpu.VMEM_SHARED((n, 2 * w), jnp.float32),
            [pltpu.SemaphoreType.DMA] * 2,
            [pltpu.SemaphoreType.DMA] * 2,
            [pltpu.SemaphoreType.DMA] * 2,
            [pltpu.SemaphoreType.DMA] * 2,
            [pltpu.SemaphoreType.DMA] * 2,
            [pltpu.SemaphoreType.DMA] * 2,
        ],
    )
    def sc_kernel(esrc_hbm, edst_hbm, bh_hbm, g_hbm, pk_hbm,
                  m_hbm, sums_hbm,
                  pk_all, isrc, idst, dstr, a_s, a_d, a_g, a_bh, comb, zbuf,
                  acc, sem_es, sem_ed, sem_bh, sem_g, sem_m, sem_sc):
        c = lax.axis_index("c")
        s = lax.axis_index("s")

        def zero_body(r, _):
            for col in range(2 * cols):
                zbuf[r, pl.ds(col * _L, _L)] = jnp.zeros((_L,), jnp.float32)
            return 0
        lax.fori_loop(0, _ZCH, zero_body, 0)

        base = s * ept
        pltpu.sync_copy(pk_hbm.at[pl.ds(base, ept)], pk_all)

        def transform(j, b, q):
            for v in range(_K // _L):
                sl = pl.ds(j * _K + v * _L, _L)
                dsl = pl.ds(v * _L, _L)
                pkv = pk_all[sl]
                sv = jnp.bitwise_and(pkv, 65535)
                dv = jnp.right_shift(pkv, 16)
                isrc[b][dsl] = sv * _NQ + q
                idst[b][dsl] = dv * _NQ + q
                dstr[b][dsl] = dv

        def issue_gathers(j, b, q):
            pltpu.async_copy(esrc_hbm.at[isrc[b]], a_s[b], sem_es[b])
            pltpu.async_copy(edst_hbm.at[idst[b]], a_d[b], sem_ed[b])
            pltpu.async_copy(bh_hbm.at[isrc[b]], a_bh[b], sem_bh[b])
            pltpu.async_copy(g_hbm.at[pl.ds(q * e + base + j * _K, _K)],
                             a_g[b], sem_g[b])

        def wait_gathers(b):
            pltpu.make_async_copy(esrc_hbm.at[isrc[b]], a_s[b],
                                  sem_es[b]).wait()
            pltpu.make_async_copy(edst_hbm.at[idst[b]], a_d[b],
                                  sem_ed[b]).wait()
            pltpu.make_async_copy(bh_hbm.at[isrc[b]], a_bh[b],
                                  sem_bh[b]).wait()
            pltpu.make_async_copy(g_hbm.at[pl.ds(0, _K)], a_g[b],
                                  sem_g[b]).wait()

        def wait_msc(b):
            pltpu.make_async_copy(a_g[b], m_hbm.at[pl.ds(0, _K)],
                                  sem_m[b]).wait()
            pltpu.make_async_copy(comb[b], acc.at[pl.ds(0, _K)],
                                  sem_sc[b]).wait()

        def compute(b):
            def row_body(r, _):
                for col in range(cols):
                    sl = pl.ds(col * _L, _L)
                    mv = a_s[b][r, sl] + a_d[b][r, sl] + a_g[b][r, sl]
                    sig = 1.0 / (1.0 + jnp.exp(-mv))
                    a_g[b][r, sl] = mv
                    comb[b][r, pl.ds(w + col * _L, _L)] = sig
                    comb[b][r, sl] = sig * a_bh[b][r, sl]
                return 0
            lax.fori_loop(0, _K, row_body, 0)

        for p in range(_NP):
            q = p * _NC + c

            @pl.when(s < _DT)
            def _():
                for i in range(_DR // _ZCH):
                    r0 = s * _DR + i * _ZCH
                    pltpu.sync_copy(zbuf, acc.at[pl.ds(r0, _ZCH)])
            plsc.subcore_barrier()

            transform(0, 0, q)
            issue_gathers(0, 0, q)

            def pair_body(t, _):
                for half in (0, 1):
                    j = 2 * t + half
                    b = half
                    nb = 1 - half
                    wait_gathers(b)

                    @pl.when(j + 1 < chunks)
                    def _():
                        @pl.when(j >= 1)
                        def _():
                            wait_msc(nb)
                        transform(j + 1, nb, q)
                        issue_gathers(j + 1, nb, q)

                    compute(b)
                    pltpu.async_copy(
                        a_g[b], m_hbm.at[pl.ds(q * e + base + j * _K, _K)],
                        sem_m[b])
                    pltpu.async_copy(comb[b], acc.at[dstr[b]], sem_sc[b],
                                     add=True)
                return 0
            lax.fori_loop(0, chunks // 2, pair_body, 0)

            wait_msc(0)
            wait_msc(1)

            plsc.subcore_barrier()

            @pl.when(s < _DT)
            def _():
                r0 = s * _DR
                pltpu.sync_copy(acc.at[pl.ds(r0, _DR)],
                                sums_hbm.at[pl.ds(q * n + r0, _DR)])
            plsc.subcore_barrier()

    return sc_kernel(esrc4, edst4, bh4, g4, pk)


def _ln_silu(v, gamma, beta):
    mu = jnp.mean(v, axis=-1, keepdims=True)
    var = jnp.mean(jnp.square(v - mu), axis=-1, keepdims=True)
    t = (v - mu) * jax.lax.rsqrt(var + 1e-5) * gamma + beta
    return t * jax.nn.sigmoid(t)


def _edge_final_body(m_ref, ef_ref, g_ref, bt_ref, y_ref):
    m = jnp.concatenate([m_ref[q] for q in range(_NQ)], axis=-1)
    y_ref[...] = ef_ref[...] + _ln_silu(m, g_ref[...], bt_ref[...])


def _edge_final(m4, edge_feats, gamma, beta, block_rows=4000):
    e, d = edge_feats.shape
    w32 = d // _NQ
    grid = (e // block_rows,)
    return pl.pallas_call(
        _edge_final_body,
        grid=grid,
        in_specs=[
            pl.BlockSpec((_NQ, block_rows, w32), lambda i: (0, i, 0)),
            pl.BlockSpec((block_rows, d), lambda i: (i, 0)),
            pl.BlockSpec((1, d), lambda i: (0, 0)),
            pl.BlockSpec((1, d), lambda i: (0, 0)),
        ],
        out_specs=pl.BlockSpec((block_rows, d), lambda i: (i, 0)),
        out_shape=jax.ShapeDtypeStruct((e, d), jnp.float32),
    )(m4, edge_feats, gamma.reshape(1, d), beta.reshape(1, d))


def _node_final_body(cx_ref, sums_ref, nf_ref, g_ref, bt_ref, x_ref):
    w32 = sums_ref.shape[-1] // 2
    ssh = jnp.concatenate([sums_ref[q, :, :w32] for q in range(_NQ)], axis=-1)
    ss = jnp.concatenate([sums_ref[q, :, w32:] for q in range(_NQ)], axis=-1)
    h = ssh / (ss + 1e-6)
    v = cx_ref[...] + h
    x_ref[...] = nf_ref[...] + _ln_silu(v, g_ref[...], bt_ref[...])


def _node_final(cx, sums4, node_feats, gamma, beta, block_rows=2000):
    n, d = cx.shape
    grid = (n // block_rows,)
    blk = pl.BlockSpec((block_rows, d), lambda i: (i, 0))
    vec = pl.BlockSpec((1, d), lambda i: (0, 0))
    return pl.pallas_call(
        _node_final_body,
        grid=grid,
        in_specs=[
            blk,
            pl.BlockSpec((_NQ, block_rows, sums4.shape[-1]),
                         lambda i: (0, i, 0)),
            blk, vec, vec,
        ],
        out_specs=blk,
        out_shape=jax.ShapeDtypeStruct((n, d), jnp.float32),
    )(cx, sums4, node_feats, gamma.reshape(1, d), beta.reshape(1, d))


def kernel(node_feats, edge_feats, edge_index,
           W_src_gate, b_src_gate, W_dst_gate, b_dst_gate,
           W_edge_gate, b_edge_gate, W_src_update, b_src_update,
           W_dst_update, b_dst_update,
           gamma_nodes, beta_nodes, gamma_edges, beta_edges):
    n, d = node_feats.shape
    w = d // _NQ
    src = edge_index[0]
    dst = edge_index[1]

    e_src, e_dst, bh, cx = _node_proj(
        node_feats,
        [W_src_gate, W_dst_gate, W_dst_update, W_src_update],
        [b_src_gate, b_dst_gate, b_dst_update, b_src_update],
    )
    g4 = _edge_proj(edge_feats, W_edge_gate, b_edge_gate)
    e = g4.shape[1]

    esrc4 = e_src.reshape(n * _NQ, w)
    edst4 = e_dst.reshape(n * _NQ, w)
    bh4 = bh.reshape(n * _NQ, w)
    pk = src + dst * 65536
    m_flat, sums_flat = _sc_middle(
        esrc4, edst4, bh4, g4.reshape(_NQ * e, w), pk, n, e, w)
    m4 = m_flat.reshape(_NQ, e, w)
    sums4 = sums_flat.reshape(_NQ, n, 2 * w)

    y = _edge_final(m4, edge_feats, gamma_edges, beta_edges)
    x = _node_final(cx, sums4, node_feats, gamma_nodes, beta_nodes)
    return (x, y)
